# baseline (speedup 1.0000x reference)
import jax
import jax.numpy as jnp
from jax import lax
from jax.experimental import pallas as pl
from jax.experimental.pallas import tpu as pltpu
from jax.experimental.pallas import tpu_sc as plsc

N, E, G, D, C = 10000, 320000, 64, 128, 10
NC, NS = 2, 16
NW = NC * NS
CH = 128
NP = 10240
ECH = 2560
EP = ECH * CH
NCHT = ECH // NW
RPS = NP // NS
GP = 72

_MESH = plsc.VectorSubcoreMesh(core_axis_name="c", subcore_axis_name="s",
                               num_cores=NC, num_subcores=NS)


BLKI = 8
NBI0, NBI1 = 19, 1
C0TOT = NS * NBI0 * BLKI


def _fill_vmem(ref, val):
    v = jnp.full((16,), val, jnp.float32)

    def row(i, carry):
        for k in range(D // 16):
            ref[i, pl.ds(k * 16, 16)] = v
        return carry

    lax.fori_loop(0, ref.shape[0], row, 0)


def _spmm_body(src_hbm, dst_hbm, u_hbm, out_hbm, acc, srcv, dstv,
               rows0, rows1, g0, g1, s0, s1):
    c = lax.axis_index("c")
    s = lax.axis_index("s")
    rb = s * RPS
    _fill_vmem(rows0, 0.0)

    def zrow(i, carry):
        pltpu.sync_copy(rows0, acc.at[pl.ds(rb + i * CH, CH)])
        return carry

    lax.fori_loop(0, RPS // CH, zrow, 0)
    plsc.subcore_barrier()

    rows = (rows0, rows1)
    gsem = (g0, g1)
    ssem = (s0, s1)
    def run(pri, nbi, cb0):
        def block(kb, carry):
            cb = cb0 + kb * BLKI
            pltpu.sync_copy(src_hbm.at[pl.ds(cb, BLKI)], srcv)
            pltpu.sync_copy(dst_hbm.at[pl.ds(cb, BLKI)], dstv)
            pltpu.async_copy(u_hbm.at[srcv.at[0]], rows0, g0, priority=pri)
            pltpu.async_copy(u_hbm.at[srcv.at[1]], rows1, g1, priority=pri)

            def pair(p, carry2):
                for b in range(2):
                    jj = 2 * p + b
                    pltpu.make_async_copy(u_hbm.at[srcv.at[jj]], rows[b],
                                          gsem[b]).wait()
                    pltpu.async_copy(rows[b], acc.at[dstv.at[jj]], ssem[b],
                                     add=True).wait()

                    @pl.when(jj + 2 < BLKI)
                    def _():
                        pltpu.async_copy(u_hbm.at[srcv.at[jj + 2]], rows[b],
                                         gsem[b], priority=pri)
                return carry2

            lax.fori_loop(0, BLKI // 2, pair, 0)
            return carry

        lax.fori_loop(0, nbi, block, 0)

    @pl.when(c == 0)
    def _():
        run(0, NBI0, s * (NBI0 * BLKI))

    @pl.when(c == 1)
    def _():
        run(1, NBI1, C0TOT + s * (NBI1 * BLKI))

    plsc.subcore_barrier()

    def orow(i, carry):
        pltpu.sync_copy(acc.at[pl.ds(rb + i * CH, CH)],
                        out_hbm.at[c, pl.ds(rb + i * CH, CH)])
        return carry

    lax.fori_loop(0, RPS // CH, orow, 0)


_spmm = pl.kernel(
    _spmm_body,
    out_type=jax.ShapeDtypeStruct((NC, NP, D), jnp.float32),
    mesh=_MESH,
    scratch_types=[
        pltpu.VMEM_SHARED((NP, D), jnp.float32),
        pltpu.VMEM((BLKI, CH), jnp.int32),
        pltpu.VMEM((BLKI, CH), jnp.int32),
        pltpu.VMEM((CH, D), jnp.float32),
        pltpu.VMEM((CH, D), jnp.float32),
        pltpu.SemaphoreType.DMA,
        pltpu.SemaphoreType.DMA,
        pltpu.SemaphoreType.DMA,
        pltpu.SemaphoreType.DMA,
    ],
)


def _deg_body(src_hbm, out_hbm, acc, srcv, onesv, zbuf, dsem):
    c = lax.axis_index("c")
    s = lax.axis_index("s")
    wid = c * NS + s
    _fill_vmem(onesv, 1.0)
    _fill_vmem(zbuf, 0.0)
    rb = s * RPS
    for i in range(RPS // CH):
        pltpu.sync_copy(zbuf, acc.at[pl.ds(rb + i * CH, CH)])
    cb = wid * NCHT
    pltpu.sync_copy(src_hbm.at[pl.ds(cb, NCHT)], srcv)
    plsc.subcore_barrier()

    FL = 4

    def blk(q, carry):
        for b in range(FL):
            pltpu.async_copy(onesv, acc.at[srcv.at[q * FL + b]], dsem,
                             add=True)
        for b in range(FL):
            pltpu.make_async_copy(onesv, acc.at[srcv.at[q * FL + b]],
                                  dsem).wait()
        return carry

    lax.fori_loop(0, NCHT // FL, blk, 0)
    plsc.subcore_barrier()
    for i in range(RPS // CH):
        pltpu.sync_copy(acc.at[pl.ds(rb + i * CH, CH)],
                        out_hbm.at[c, pl.ds(rb + i * CH, CH)])


_deg = pl.kernel(
    _deg_body,
    out_type=jax.ShapeDtypeStruct((NC, NP, D), jnp.float32),
    mesh=_MESH,
    scratch_types=[
        pltpu.VMEM_SHARED((NP, D), jnp.float32),
        pltpu.VMEM((NCHT, CH), jnp.int32),
        pltpu.VMEM((CH, D), jnp.float32),
        pltpu.VMEM((CH, D), jnp.float32),
        pltpu.SemaphoreType.DMA,
    ],
)


NPCH = NP // CH
PK = -(-NPCH // NW)


def _pool_body(bidx_hbm, h_hbm, z_hbm, ones_hbm, pout, cout,
               pacc, cacc, bidx, rows, onesv):
    c = lax.axis_index("c")
    s = lax.axis_index("s")
    wid = c * NS + s
    pltpu.sync_copy(ones_hbm, onesv)

    @pl.when(s == 0)
    def _():
        pltpu.sync_copy(z_hbm.at[pl.ds(0, GP)], pacc)
        pltpu.sync_copy(z_hbm.at[pl.ds(0, GP)], cacc)

    pltpu.sync_copy(bidx_hbm, bidx)
    plsc.subcore_barrier()

    for k in range(PK):
        cj = wid + k * NW

        @pl.when(cj < NPCH)
        def _():
            pltpu.sync_copy(h_hbm.at[pl.ds(cj * CH, CH)], rows)
            pltpu.sync_copy(rows, pacc.at[bidx.at[cj]], add=True)
            pltpu.sync_copy(onesv, cacc.at[bidx.at[cj]], add=True)

    plsc.subcore_barrier()

    @pl.when(s == 0)
    def _():
        pltpu.sync_copy(pacc, pout.at[c])
        pltpu.sync_copy(cacc, cout.at[c])


_pool = pl.kernel(
    _pool_body,
    out_type=(jax.ShapeDtypeStruct((NC, GP, D), jnp.float32),
              jax.ShapeDtypeStruct((NC, GP, D), jnp.float32)),
    mesh=_MESH,
    scratch_types=[
        pltpu.VMEM_SHARED((GP, D), jnp.float32),
        pltpu.VMEM_SHARED((GP, D), jnp.float32),
        pltpu.VMEM((NPCH, CH), jnp.int32),
        pltpu.VMEM((CH, D), jnp.float32),
        pltpu.VMEM((CH, D), jnp.float32),
    ],
)


BR = 512
NBLK = NP // BR


def _u0_body(dacc_ref, x_ref, disb_ref, u_ref):
    deg = dacc_ref[0, :, 0] + dacc_ref[1, :, 0]
    dis = jnp.where(deg > 0.0, lax.rsqrt(jnp.maximum(deg, 1e-30)), 0.0)
    db = jnp.broadcast_to(dis[:, None], (BR, D))
    disb_ref[...] = db
    u_ref[...] = x_ref[...] * db


_u0 = pl.pallas_call(
    _u0_body,
    grid=(NBLK,),
    in_specs=[pl.BlockSpec((NC, BR, D), lambda i: (0, i, 0)),
              pl.BlockSpec((BR, D), lambda i: (i, 0))],
    out_specs=[pl.BlockSpec((BR, D), lambda i: (i, 0)),
               pl.BlockSpec((BR, D), lambda i: (i, 0))],
    out_shape=[jax.ShapeDtypeStruct((NP, D), jnp.float32),
               jax.ShapeDtypeStruct((NP, D), jnp.float32)],
)


def _bmid_body(s_ref, disb_ref, y1_ref, u2_ref):
    db = disb_ref[...]
    y1 = db * (s_ref[0] + s_ref[1])
    y1_ref[...] = y1
    u2_ref[...] = db * y1


_bmid = pl.pallas_call(
    _bmid_body,
    grid=(NBLK,),
    in_specs=[pl.BlockSpec((NC, BR, D), lambda i: (0, i, 0)),
              pl.BlockSpec((BR, D), lambda i: (i, 0))],
    out_specs=[pl.BlockSpec((BR, D), lambda i: (i, 0)),
               pl.BlockSpec((BR, D), lambda i: (i, 0))],
    out_shape=[jax.ShapeDtypeStruct((NP, D), jnp.float32),
               jax.ShapeDtypeStruct((NP, D), jnp.float32)],
)


def _make_clayer(relu, with_u):
    def body(h_ref, y1_ref, s2_ref, disb_ref, w0_ref, w1_ref, w2_ref, b_ref,
             *outs):
        db = disb_ref[...]
        h = h_ref[...]
        y2 = db * (s2_ref[0] + s2_ref[1])
        t2 = 2.0 * y2 - h
        z = jnp.dot(h, w0_ref[...], preferred_element_type=jnp.float32)
        z = z - jnp.dot(y1_ref[...], w1_ref[...],
                        preferred_element_type=jnp.float32)
        z = z + jnp.dot(t2, w2_ref[...], preferred_element_type=jnp.float32)
        z = z + b_ref[...]
        if relu:
            z = jnp.maximum(z, 0.0)
        outs[0][...] = z
        if with_u:
            outs[1][...] = db * z

    n_out = 2 if with_u else 1
    return pl.pallas_call(
        body,
        grid=(NBLK,),
        in_specs=[pl.BlockSpec((BR, D), lambda i: (i, 0)),
                  pl.BlockSpec((BR, D), lambda i: (i, 0)),
                  pl.BlockSpec((NC, BR, D), lambda i: (0, i, 0)),
                  pl.BlockSpec((BR, D), lambda i: (i, 0)),
                  pl.BlockSpec((D, D), lambda i: (0, 0)),
                  pl.BlockSpec((D, D), lambda i: (0, 0)),
                  pl.BlockSpec((D, D), lambda i: (0, 0)),
                  pl.BlockSpec((1, D), lambda i: (0, 0))],
        out_specs=[pl.BlockSpec((BR, D), lambda i: (i, 0))] * n_out,
        out_shape=[jax.ShapeDtypeStruct((NP, D), jnp.float32)] * n_out,
    )


_c_mid = _make_clayer(relu=True, with_u=True)
_c_last = _make_clayer(relu=False, with_u=False)


def _f_body(pacc_ref, cacc_ref, wlin_ref, blin_ref, out_ref):
    p = pacc_ref[0, :G, :] + pacc_ref[1, :G, :]
    cnt = cacc_ref[0, :G, 0] + cacc_ref[1, :G, 0]
    pooled = p / jnp.maximum(cnt, 1.0)[:, None]
    out_ref[...] = (jnp.dot(pooled, wlin_ref[...],
                            preferred_element_type=jnp.float32)
                    + blin_ref[...])


_f = pl.pallas_call(
    _f_body,
    out_shape=jax.ShapeDtypeStruct((G, C), jnp.float32),
)


def kernel(x, edge_index, batch, W1, b1, W2, b2, W3, b3, Wlin, blin):
    f32 = jnp.float32
    src = edge_index[0]
    dst = edge_index[1]
    padN = jnp.full((EP - E,), N, jnp.int32)
    src_p = jnp.concatenate([src, padN]).reshape(ECH, CH)
    dst_p = jnp.concatenate([dst, padN]).reshape(ECH, CH)
    x_p = jnp.concatenate([x, jnp.zeros((NP - N, D), f32)], axis=0)
    batch_p = jnp.concatenate(
        [batch, jnp.full((NP - N,), G, jnp.int32)]).reshape(NPCH, CH)
    zrows = jnp.zeros((CH, D), f32)
    orows = jnp.ones((CH, D), f32)

    dacc = _deg(src_p)
    disb, u = _u0(dacc, x_p)
    h = x_p
    for (W, b, last) in ((W1, b1, False), (W2, b2, False), (W3, b3, True)):
        s1 = _spmm(src_p, dst_p, u)
        y1, u2 = _bmid(s1, disb)
        s2 = _spmm(src_p, dst_p, u2)
        b2d = b.reshape(1, D)
        if last:
            (h,) = _c_last(h, y1, s2, disb, W[0], W[1], W[2], b2d)
        else:
            h, u = _c_mid(h, y1, s2, disb, W[0], W[1], W[2], b2d)
    pacc, cacc = _pool(batch_p, h, zrows, orows)
    return _f(pacc, cacc, Wlin, blin.reshape(1, C))


# speedup vs baseline: 1.0011x; 1.0011x over previous
import jax
import jax.numpy as jnp
from jax import lax
from jax.experimental import pallas as pl
from jax.experimental.pallas import tpu as pltpu
from jax.experimental.pallas import tpu_sc as plsc

N, E, G, D, C = 10000, 320000, 64, 128, 10
NC, NS = 2, 16
NW = NC * NS
CH = 128
NP = 10240
ECH = 2560
EP = ECH * CH
NCHT = ECH // NW
RPS = NP // NS
GP = 72

_MESH = plsc.VectorSubcoreMesh(core_axis_name="c", subcore_axis_name="s",
                               num_cores=NC, num_subcores=NS)


BLKI = 8
NBI0, NBI1 = 19, 1
C0TOT = NS * NBI0 * BLKI


def _fill_vmem(ref, val):
    v = jnp.full((16,), val, jnp.float32)

    def row(i, carry):
        for k in range(D // 16):
            ref[i, pl.ds(k * 16, 16)] = v
        return carry

    lax.fori_loop(0, ref.shape[0], row, 0)


def _spmm_body(src_hbm, dst_hbm, u_hbm, out_hbm, acc, srcv, dstv,
               rows0, rows1, g0, g1, s0, s1):
    c = lax.axis_index("c")
    s = lax.axis_index("s")
    rb = s * RPS
    _fill_vmem(rows0, 0.0)

    def zrow(i, carry):
        pltpu.sync_copy(rows0, acc.at[pl.ds(rb + i * CH, CH)])
        return carry

    lax.fori_loop(0, RPS // CH, zrow, 0)
    plsc.subcore_barrier()

    rows = (rows0, rows1)
    gsem = (g0, g1)
    ssem = (s0, s1)
    nbi = jnp.where(c == 0, NBI0, NBI1)
    cb0 = jnp.where(c == 0, s * (NBI0 * BLKI), C0TOT + s * (NBI1 * BLKI))

    def block(kb, carry):
        cb = cb0 + kb * BLKI
        pltpu.sync_copy(src_hbm.at[pl.ds(cb, BLKI)], srcv)
        pltpu.sync_copy(dst_hbm.at[pl.ds(cb, BLKI)], dstv)
        pltpu.async_copy(u_hbm.at[srcv.at[0]], rows0, g0)
        pltpu.async_copy(u_hbm.at[srcv.at[1]], rows1, g1)

        def pair(p, carry2):
            for b in range(2):
                jj = 2 * p + b
                pltpu.make_async_copy(u_hbm.at[srcv.at[jj]], rows[b],
                                      gsem[b]).wait()
                pltpu.async_copy(rows[b], acc.at[dstv.at[jj]], ssem[b],
                                 add=True).wait()

                @pl.when(jj + 2 < BLKI)
                def _():
                    pltpu.async_copy(u_hbm.at[srcv.at[jj + 2]], rows[b],
                                     gsem[b])
            return carry2

        lax.fori_loop(0, BLKI // 2, pair, 0)
        return carry

    lax.fori_loop(0, nbi, block, 0)
    plsc.subcore_barrier()

    def orow(i, carry):
        pltpu.sync_copy(acc.at[pl.ds(rb + i * CH, CH)],
                        out_hbm.at[c, pl.ds(rb + i * CH, CH)])
        return carry

    lax.fori_loop(0, RPS // CH, orow, 0)


_spmm = pl.kernel(
    _spmm_body,
    out_type=jax.ShapeDtypeStruct((NC, NP, D), jnp.float32),
    mesh=_MESH,
    scratch_types=[
        pltpu.VMEM_SHARED((NP, D), jnp.float32),
        pltpu.VMEM((BLKI, CH), jnp.int32),
        pltpu.VMEM((BLKI, CH), jnp.int32),
        pltpu.VMEM((CH, D), jnp.float32),
        pltpu.VMEM((CH, D), jnp.float32),
        pltpu.SemaphoreType.DMA,
        pltpu.SemaphoreType.DMA,
        pltpu.SemaphoreType.DMA,
        pltpu.SemaphoreType.DMA,
    ],
)


def _deg_body(src_hbm, out_hbm, acc, srcv, onesv, zbuf, dsem):
    c = lax.axis_index("c")
    s = lax.axis_index("s")
    wid = c * NS + s
    _fill_vmem(onesv, 1.0)
    _fill_vmem(zbuf, 0.0)
    rb = s * RPS
    for i in range(RPS // CH):
        pltpu.sync_copy(zbuf, acc.at[pl.ds(rb + i * CH, CH)])
    cb = wid * NCHT
    pltpu.sync_copy(src_hbm.at[pl.ds(cb, NCHT)], srcv)
    plsc.subcore_barrier()

    FL = 4

    def blk(q, carry):
        for b in range(FL):
            pltpu.async_copy(onesv, acc.at[srcv.at[q * FL + b]], dsem,
                             add=True)
        for b in range(FL):
            pltpu.make_async_copy(onesv, acc.at[srcv.at[q * FL + b]],
                                  dsem).wait()
        return carry

    lax.fori_loop(0, NCHT // FL, blk, 0)
    plsc.subcore_barrier()
    for i in range(RPS // CH):
        pltpu.sync_copy(acc.at[pl.ds(rb + i * CH, CH)],
                        out_hbm.at[c, pl.ds(rb + i * CH, CH)])


_deg = pl.kernel(
    _deg_body,
    out_type=jax.ShapeDtypeStruct((NC, NP, D), jnp.float32),
    mesh=_MESH,
    scratch_types=[
        pltpu.VMEM_SHARED((NP, D), jnp.float32),
        pltpu.VMEM((NCHT, CH), jnp.int32),
        pltpu.VMEM((CH, D), jnp.float32),
        pltpu.VMEM((CH, D), jnp.float32),
        pltpu.SemaphoreType.DMA,
    ],
)


NPCH = NP // CH
PK = -(-NPCH // NW)


def _pool_body(bidx_hbm, h_hbm, z_hbm, ones_hbm, pout, cout,
               pacc, cacc, bidx, rows, onesv):
    c = lax.axis_index("c")
    s = lax.axis_index("s")
    wid = c * NS + s
    pltpu.sync_copy(ones_hbm, onesv)

    @pl.when(s == 0)
    def _():
        pltpu.sync_copy(z_hbm.at[pl.ds(0, GP)], pacc)
        pltpu.sync_copy(z_hbm.at[pl.ds(0, GP)], cacc)

    pltpu.sync_copy(bidx_hbm, bidx)
    plsc.subcore_barrier()

    for k in range(PK):
        cj = wid + k * NW

        @pl.when(cj < NPCH)
        def _():
            pltpu.sync_copy(h_hbm.at[pl.ds(cj * CH, CH)], rows)
            pltpu.sync_copy(rows, pacc.at[bidx.at[cj]], add=True)
            pltpu.sync_copy(onesv, cacc.at[bidx.at[cj]], add=True)

    plsc.subcore_barrier()

    @pl.when(s == 0)
    def _():
        pltpu.sync_copy(pacc, pout.at[c])
        pltpu.sync_copy(cacc, cout.at[c])


_pool = pl.kernel(
    _pool_body,
    out_type=(jax.ShapeDtypeStruct((NC, GP, D), jnp.float32),
              jax.ShapeDtypeStruct((NC, GP, D), jnp.float32)),
    mesh=_MESH,
    scratch_types=[
        pltpu.VMEM_SHARED((GP, D), jnp.float32),
        pltpu.VMEM_SHARED((GP, D), jnp.float32),
        pltpu.VMEM((NPCH, CH), jnp.int32),
        pltpu.VMEM((CH, D), jnp.float32),
        pltpu.VMEM((CH, D), jnp.float32),
    ],
)


BR = 512
NBLK = NP // BR


def _u0_body(dacc_ref, x_ref, disb_ref, u_ref):
    deg = dacc_ref[0, :, 0] + dacc_ref[1, :, 0]
    dis = jnp.where(deg > 0.0, lax.rsqrt(jnp.maximum(deg, 1e-30)), 0.0)
    db = jnp.broadcast_to(dis[:, None], (BR, D))
    disb_ref[...] = db
    u_ref[...] = x_ref[...] * db


_u0 = pl.pallas_call(
    _u0_body,
    grid=(NBLK,),
    in_specs=[pl.BlockSpec((NC, BR, D), lambda i: (0, i, 0)),
              pl.BlockSpec((BR, D), lambda i: (i, 0))],
    out_specs=[pl.BlockSpec((BR, D), lambda i: (i, 0)),
               pl.BlockSpec((BR, D), lambda i: (i, 0))],
    out_shape=[jax.ShapeDtypeStruct((NP, D), jnp.float32),
               jax.ShapeDtypeStruct((NP, D), jnp.float32)],
)


def _bmid_body(s_ref, disb_ref, y1_ref, u2_ref):
    db = disb_ref[...]
    y1 = db * (s_ref[0] + s_ref[1])
    y1_ref[...] = y1
    u2_ref[...] = db * y1


_bmid = pl.pallas_call(
    _bmid_body,
    grid=(NBLK,),
    in_specs=[pl.BlockSpec((NC, BR, D), lambda i: (0, i, 0)),
              pl.BlockSpec((BR, D), lambda i: (i, 0))],
    out_specs=[pl.BlockSpec((BR, D), lambda i: (i, 0)),
               pl.BlockSpec((BR, D), lambda i: (i, 0))],
    out_shape=[jax.ShapeDtypeStruct((NP, D), jnp.float32),
               jax.ShapeDtypeStruct((NP, D), jnp.float32)],
)


def _make_clayer(relu, with_u):
    def body(h_ref, y1_ref, s2_ref, disb_ref, w0_ref, w1_ref, w2_ref, b_ref,
             *outs):
        db = disb_ref[...]
        h = h_ref[...]
        y2 = db * (s2_ref[0] + s2_ref[1])
        t2 = 2.0 * y2 - h
        z = jnp.dot(h, w0_ref[...], preferred_element_type=jnp.float32)
        z = z - jnp.dot(y1_ref[...], w1_ref[...],
                        preferred_element_type=jnp.float32)
        z = z + jnp.dot(t2, w2_ref[...], preferred_element_type=jnp.float32)
        z = z + b_ref[...]
        if relu:
            z = jnp.maximum(z, 0.0)
        outs[0][...] = z
        if with_u:
            outs[1][...] = db * z

    n_out = 2 if with_u else 1
    return pl.pallas_call(
        body,
        grid=(NBLK,),
        in_specs=[pl.BlockSpec((BR, D), lambda i: (i, 0)),
                  pl.BlockSpec((BR, D), lambda i: (i, 0)),
                  pl.BlockSpec((NC, BR, D), lambda i: (0, i, 0)),
                  pl.BlockSpec((BR, D), lambda i: (i, 0)),
                  pl.BlockSpec((D, D), lambda i: (0, 0)),
                  pl.BlockSpec((D, D), lambda i: (0, 0)),
                  pl.BlockSpec((D, D), lambda i: (0, 0)),
                  pl.BlockSpec((1, D), lambda i: (0, 0))],
        out_specs=[pl.BlockSpec((BR, D), lambda i: (i, 0))] * n_out,
        out_shape=[jax.ShapeDtypeStruct((NP, D), jnp.float32)] * n_out,
    )


_c_mid = _make_clayer(relu=True, with_u=True)
_c_last = _make_clayer(relu=False, with_u=False)


def _f_body(pacc_ref, cacc_ref, wlin_ref, blin_ref, out_ref):
    p = pacc_ref[0, :G, :] + pacc_ref[1, :G, :]
    cnt = cacc_ref[0, :G, 0] + cacc_ref[1, :G, 0]
    pooled = p / jnp.maximum(cnt, 1.0)[:, None]
    out_ref[...] = (jnp.dot(pooled, wlin_ref[...],
                            preferred_element_type=jnp.float32)
                    + blin_ref[...])


_f = pl.pallas_call(
    _f_body,
    out_shape=jax.ShapeDtypeStruct((G, C), jnp.float32),
)


def kernel(x, edge_index, batch, W1, b1, W2, b2, W3, b3, Wlin, blin):
    f32 = jnp.float32
    src = edge_index[0]
    dst = edge_index[1]
    padN = jnp.full((EP - E,), N, jnp.int32)
    src_p = jnp.concatenate([src, padN]).reshape(ECH, CH)
    dst_p = jnp.concatenate([dst, padN]).reshape(ECH, CH)
    x_p = jnp.concatenate([x, jnp.zeros((NP - N, D), f32)], axis=0)
    batch_p = jnp.concatenate(
        [batch, jnp.full((NP - N,), G, jnp.int32)]).reshape(NPCH, CH)
    zrows = jnp.zeros((CH, D), f32)
    orows = jnp.ones((CH, D), f32)

    dacc = _deg(src_p)
    disb, u = _u0(dacc, x_p)
    h = x_p
    for (W, b, last) in ((W1, b1, False), (W2, b2, False), (W3, b3, True)):
        s1 = _spmm(src_p, dst_p, u)
        y1, u2 = _bmid(s1, disb)
        s2 = _spmm(src_p, dst_p, u2)
        b2d = b.reshape(1, D)
        if last:
            (h,) = _c_last(h, y1, s2, disb, W[0], W[1], W[2], b2d)
        else:
            h, u = _c_mid(h, y1, s2, disb, W[0], W[1], W[2], b2d)
    pacc, cacc = _pool(batch_p, h, zrows, orows)
    return _f(pacc, cacc, Wlin, blin.reshape(1, C))
